# baseline (speedup 1.0000x reference)
import functools

import jax
import jax.numpy as jnp
from jax import lax
from jax.experimental import pallas as pl
from jax.experimental.pallas import tpu as pltpu
from jax.experimental.pallas import tpu_sc as plsc

B = 16384
D = 32
NC = 2
NS = 16
NW = NC * NS
BPW = B // NW
L = 16
ROW_UNROLL = 8

_mesh = plsc.VectorSubcoreMesh(core_axis_name="c", subcore_axis_name="s")


@functools.partial(
    pl.kernel,
    mesh=_mesh,
    out_type=jax.ShapeDtypeStruct((B, D), jnp.float32),
    compiler_params=pltpu.CompilerParams(use_tc_tiling_on_sc=False),
    scratch_types=[
        pltpu.VMEM((BPW,), jnp.int32),
        pltpu.VMEM((BPW,), jnp.int32),
        pltpu.VMEM((BPW, D), jnp.float32),
        pltpu.VMEM((BPW, D), jnp.float32),
        pltpu.SemaphoreType.DMA,
        pltpu.SemaphoreType.DMA,
    ],
)
def _gmf(users_hbm, items_hbm, uemb_hbm, iemb_hbm, out_hbm,
         uidx_v, iidx_v, urows_v, irows_v, sem_u, sem_i):
    wid = lax.axis_index("s") * NC + lax.axis_index("c")
    base = wid * BPW

    pltpu.sync_copy(users_hbm.at[pl.ds(base, BPW)], uidx_v)
    pltpu.sync_copy(items_hbm.at[pl.ds(base, BPW)], iidx_v)

    cu = pltpu.async_copy(uemb_hbm.at[uidx_v], urows_v, sem_u)
    ci = pltpu.async_copy(iemb_hbm.at[iidx_v], irows_v, sem_i)
    cu.wait()
    ci.wait()

    def body(rb, carry):
        for dr in range(ROW_UNROLL):
            r = rb * ROW_UNROLL + dr
            for c in range(D // L):
                sl = pl.ds(c * L, L)
                u = urows_v[r, sl]
                i = irows_v[r, sl]
                urows_v[r, sl] = (u + 1.0) * (i + 1.0) * 0.25
        return carry

    lax.fori_loop(0, BPW // ROW_UNROLL, body, 0)

    pltpu.sync_copy(urows_v, out_hbm.at[pl.ds(base, BPW)])


def kernel(x, user_emb, item_emb):
    xi = x.astype(jnp.int32)
    users = xi[:, 0]
    items = xi[:, 1]
    return _gmf(users, items, user_emb, item_emb)


# speedup vs baseline: 1.4884x; 1.4884x over previous
import functools

import jax
import jax.numpy as jnp
from jax import lax
from jax.experimental import pallas as pl
from jax.experimental.pallas import tpu as pltpu
from jax.experimental.pallas import tpu_sc as plsc

B = 16384
D = 32
NC = 2
NS = 16
NW = NC * NS
BPW = B // NW
CH = 256
NCH = BPW // CH
L = 16

_mesh = plsc.VectorSubcoreMesh(core_axis_name="c", subcore_axis_name="s")


@functools.partial(
    pl.kernel,
    mesh=_mesh,
    out_type=jax.ShapeDtypeStruct((B, D), jnp.float32),
    scratch_types=[
        pltpu.VMEM((BPW,), jnp.int32),
        pltpu.VMEM((BPW,), jnp.int32),
        pltpu.VMEM((CH, D), jnp.float32),
        pltpu.VMEM((CH, D), jnp.float32),
        pltpu.SemaphoreType.DMA,
        pltpu.SemaphoreType.DMA,
    ],
)
def _gmf(users_hbm, items_hbm, uemb_hbm, iemb_hbm, out_hbm,
         uidx_v, iidx_v, urows, irows, sem_u, sem_i):
    wid = lax.axis_index("s") * NC + lax.axis_index("c")
    base = wid * BPW

    pltpu.sync_copy(users_hbm.at[pl.ds(base, BPW)], uidx_v)
    pltpu.sync_copy(items_hbm.at[pl.ds(base, BPW)], iidx_v)

    for ch in range(NCH):
        cbase = base + ch * CH

        def issue(g, carry):
            uvec = uidx_v[pl.ds(ch * CH + g * L, L)]
            ivec = iidx_v[pl.ds(ch * CH + g * L, L)]
            for k in range(L):
                rr = g * L + k
                pltpu.async_copy(uemb_hbm.at[uvec[k]], urows.at[rr], sem_u)
                pltpu.async_copy(iemb_hbm.at[ivec[k]], irows.at[rr], sem_i)
            return carry

        lax.fori_loop(0, CH // L, issue, 0)

        pltpu.make_async_copy(uemb_hbm.at[pl.ds(0, CH)], urows, sem_u).wait()
        pltpu.make_async_copy(iemb_hbm.at[pl.ds(0, CH)], irows, sem_i).wait()

        def mul(r, carry):
            for h in range(D // L):
                sl = pl.ds(h * L, L)
                u = urows[r, sl]
                i = irows[r, sl]
                urows[r, sl] = (u + 1.0) * (i + 1.0) * 0.25
            return carry

        lax.fori_loop(0, CH, mul, 0)

        pltpu.sync_copy(urows, out_hbm.at[pl.ds(cbase, CH)])


def kernel(x, user_emb, item_emb):
    xi = x.astype(jnp.int32)
    return _gmf(xi[:, 0], xi[:, 1], user_emb, item_emb)


# speedup vs baseline: 3.6167x; 2.4299x over previous
import functools

import jax
import jax.numpy as jnp
from jax import lax
from jax.experimental import pallas as pl
from jax.experimental.pallas import tpu as pltpu
from jax.experimental.pallas import tpu_sc as plsc

B = 16384
D = 32
NC = 2
NS = 16
NW = NC * NS
BPW = B // NW
L = 16
RING = 8
LANES = 128
NG = BPW // L

_mesh = plsc.VectorSubcoreMesh(core_axis_name="c", subcore_axis_name="s")

_scratch = [
    pltpu.VMEM((BPW,), jnp.int32),
    pltpu.VMEM((BPW,), jnp.int32),
    pltpu.VMEM((D, BPW), jnp.float32),
]
_scratch += [pltpu.VMEM((D, LANES), jnp.float32) for _ in range(2 * RING)]
_scratch += [pltpu.SemaphoreType.DMA for _ in range(2 * RING)]


@functools.partial(
    pl.kernel,
    mesh=_mesh,
    out_type=jax.ShapeDtypeStruct((D, B), jnp.float32),
    compiler_params=pltpu.CompilerParams(needs_layout_passes=False),
    scratch_types=_scratch,
)
def _gmf(users_hbm, items_hbm, uembT_hbm, iembT_hbm, outT_hbm,
         uidx_v, iidx_v, obuf, *ring):
    ublk = ring[0:RING]
    iblk = ring[RING:2 * RING]
    usem = ring[2 * RING:3 * RING]
    isem = ring[3 * RING:4 * RING]

    wid = lax.axis_index("s") * NC + lax.axis_index("c")
    base = wid * BPW

    pltpu.sync_copy(users_hbm.at[pl.ds(base, BPW)], uidx_v)
    pltpu.sync_copy(items_hbm.at[pl.ds(base, BPW)], iidx_v)

    rows0 = lax.iota(jnp.int32, L)

    def body(g, carry):
        uvec = uidx_v[pl.ds(g * L, L)]
        ivec = iidx_v[pl.ds(g * L, L)]

        def fetch(k, j):
            uo = pl.multiple_of((uvec[k] >> 7) << 7, LANES)
            io = pl.multiple_of((ivec[k] >> 7) << 7, LANES)
            pltpu.async_copy(uembT_hbm.at[:, pl.ds(uo, LANES)],
                             ublk[j], usem[j])
            pltpu.async_copy(iembT_hbm.at[:, pl.ds(io, LANES)],
                             iblk[j], isem[j])

        for j in range(RING):
            fetch(j, j)

        for k in range(L):
            j = k % RING
            ul = jnp.full((L,), uvec[k] & (LANES - 1), jnp.int32)
            il = jnp.full((L,), ivec[k] & (LANES - 1), jnp.int32)
            pltpu.make_async_copy(
                uembT_hbm.at[:, pl.ds(0, LANES)], ublk[j], usem[j]).wait()
            pltpu.make_async_copy(
                iembT_hbm.at[:, pl.ds(0, LANES)], iblk[j], isem[j]).wait()
            rcol = jnp.full((L,), g * L + k, jnp.int32)
            for h in range(D // L):
                u = plsc.load_gather(ublk[j], [rows0 + h * L, ul])
                i = plsc.load_gather(iblk[j], [rows0 + h * L, il])
                plsc.store_scatter(obuf, [rows0 + h * L, rcol],
                                   (u + 1.0) * (i + 1.0) * 0.25)
            if k < RING:
                fetch(k + RING, j)
        return carry

    lax.fori_loop(0, NG, body, 0)

    pltpu.sync_copy(obuf, outT_hbm.at[:, pl.ds(base, BPW)])


def kernel(x, user_emb, item_emb):
    xi = x.astype(jnp.int32)
    outT = _gmf(xi[:, 0], xi[:, 1], user_emb.T, item_emb.T)
    return outT.T


# speedup vs baseline: 3.9347x; 1.0879x over previous
import functools

import jax
import jax.numpy as jnp
from jax import lax
from jax.experimental import pallas as pl
from jax.experimental.pallas import tpu as pltpu
from jax.experimental.pallas import tpu_sc as plsc

B = 16384
D = 32
NC = 2
NS = 16
NW = NC * NS
BPW = B // NW
L = 16
RING = 8
LANES = 128
NG = BPW // L

_mesh = plsc.VectorSubcoreMesh(core_axis_name="c", subcore_axis_name="s")

_scratch = [
    pltpu.VMEM((BPW,), jnp.int32),
    pltpu.VMEM((BPW,), jnp.int32),
    pltpu.VMEM((D, BPW), jnp.float32),
]
_scratch += [pltpu.VMEM((D, LANES), jnp.float32) for _ in range(2 * RING)]
_scratch += [pltpu.SemaphoreType.DMA for _ in range(2 * RING)]


@functools.partial(
    pl.kernel,
    mesh=_mesh,
    out_type=jax.ShapeDtypeStruct((D, B), jnp.float32),
    compiler_params=pltpu.CompilerParams(needs_layout_passes=False),
    scratch_types=_scratch,
)
def _gmf(users_hbm, items_hbm, uembT_hbm, iembT_hbm, outT_hbm,
         uidx_v, iidx_v, obuf, *ring):
    ublk = ring[0:RING]
    iblk = ring[RING:2 * RING]
    usem = ring[2 * RING:3 * RING]
    isem = ring[3 * RING:4 * RING]

    wid = lax.axis_index("s") * NC + lax.axis_index("c")
    base = wid * BPW

    pltpu.sync_copy(users_hbm.at[pl.ds(base, BPW)], uidx_v)
    pltpu.sync_copy(items_hbm.at[pl.ds(base, BPW)], iidx_v)

    rows0 = lax.iota(jnp.int32, L)

    def fetch(uvec, ivec, k, j):
        uo = pl.multiple_of((uvec[k] >> 7) << 7, LANES)
        io = pl.multiple_of((ivec[k] >> 7) << 7, LANES)
        pltpu.async_copy(uembT_hbm.at[:, pl.ds(uo, LANES)],
                         ublk[j], usem[j])
        pltpu.async_copy(iembT_hbm.at[:, pl.ds(io, LANES)],
                         iblk[j], isem[j])

    uvec0 = uidx_v[pl.ds(0, L)]
    ivec0 = iidx_v[pl.ds(0, L)]
    for j in range(RING):
        fetch(uvec0, ivec0, j, j)

    def body(g, carry):
        uvec = uidx_v[pl.ds(g * L, L)]
        ivec = iidx_v[pl.ds(g * L, L)]
        gn = jnp.minimum(g + 1, NG - 1)
        uvec_n = uidx_v[pl.ds(gn * L, L)]
        ivec_n = iidx_v[pl.ds(gn * L, L)]

        for k in range(L):
            j = k % RING
            ul = jnp.full((L,), uvec[k] & (LANES - 1), jnp.int32)
            il = jnp.full((L,), ivec[k] & (LANES - 1), jnp.int32)
            pltpu.make_async_copy(
                uembT_hbm.at[:, pl.ds(0, LANES)], ublk[j], usem[j]).wait()
            pltpu.make_async_copy(
                iembT_hbm.at[:, pl.ds(0, LANES)], iblk[j], isem[j]).wait()
            rcol = jnp.full((L,), g * L + k, jnp.int32)
            for h in range(D // L):
                u = plsc.load_gather(ublk[j], [rows0 + h * L, ul])
                i = plsc.load_gather(iblk[j], [rows0 + h * L, il])
                plsc.store_scatter(obuf, [rows0 + h * L, rcol],
                                   (u + 1.0) * (i + 1.0) * 0.25)
            if k < L - RING:
                fetch(uvec, ivec, k + RING, j)
            else:
                @pl.when(g < NG - 1)
                def _():
                    fetch(uvec_n, ivec_n, k + RING - L, j)
        return carry

    lax.fori_loop(0, NG, body, 0)

    pltpu.sync_copy(obuf, outT_hbm.at[:, pl.ds(base, BPW)])


def kernel(x, user_emb, item_emb):
    xi = x.astype(jnp.int32)
    outT = _gmf(xi[:, 0], xi[:, 1], user_emb.T, item_emb.T)
    return outT.T
